# baseline (speedup 1.0000x reference)
import functools

import jax
import jax.numpy as jnp
from jax import lax
from jax.experimental import pallas as pl
from jax.experimental.pallas import tpu as pltpu
from jax.experimental.pallas import tpu_sc as plsc

F = 256
FH = 128


def _lrelu(t):
    return jnp.where(t >= 0, t, 0.01 * t)


def _mm_bias_act_kernel(x_ref, w_ref, b_ref, o_ref, *, act):
    o_ref[...] = act(
        jnp.dot(x_ref[...], w_ref[...], preferred_element_type=jnp.float32)
        + b_ref[...])


def _mm_bias_act(x, w, b, act, block_rows):
    rows, k = x.shape
    m = w.shape[1]
    return pl.pallas_call(
        functools.partial(_mm_bias_act_kernel, act=act),
        grid=(rows // block_rows,),
        in_specs=[
            pl.BlockSpec((block_rows, k), lambda i: (i, 0)),
            pl.BlockSpec((k, m), lambda i: (0, 0)),
            pl.BlockSpec((1, m), lambda i: (0, 0)),
        ],
        out_specs=pl.BlockSpec((block_rows, m), lambda i: (i, 0)),
        out_shape=jax.ShapeDtypeStruct((rows, m), jnp.float32),
    )(x, w, b)


def _edge_precompute_kernel(ea_ref, we_ref, be_ref, wall_ref, ball_ref,
                            *o_refs, nplanes):
    ea = _lrelu(
        jnp.dot(ea_ref[...], we_ref[...], preferred_element_type=jnp.float32)
        + be_ref[...])
    val = (jnp.dot(ea, wall_ref[...], preferred_element_type=jnp.float32)
           + ball_ref[...])
    for j in range(nplanes):
        o_refs[j][...] = val[:, j * F:(j + 1) * F]


def _edge_precompute(edge_attr_p, w_e, b_e, w_all, b_all, nplanes,
                     block_rows=640):
    e, k = edge_attr_p.shape
    m = w_all.shape[1]
    return pl.pallas_call(
        functools.partial(_edge_precompute_kernel, nplanes=nplanes),
        grid=(e // block_rows,),
        in_specs=[
            pl.BlockSpec((block_rows, k), lambda i: (i, 0)),
            pl.BlockSpec((k, F), lambda i: (0, 0)),
            pl.BlockSpec((1, F), lambda i: (0, 0)),
            pl.BlockSpec((F, m), lambda i: (0, 0)),
            pl.BlockSpec((1, m), lambda i: (0, 0)),
        ],
        out_specs=[pl.BlockSpec((block_rows, F), lambda i: (i, 0))
                   for _ in range(nplanes)],
        out_shape=[jax.ShapeDtypeStruct((e, F), jnp.float32)
                   for _ in range(nplanes)],
    )(edge_attr_p, w_e, b_e, w_all, b_all)


def _proj_kernel(h_ref, w_ref, o0_ref, o1_ref):
    val = jnp.dot(h_ref[...], w_ref[...], preferred_element_type=jnp.float32)
    for side in range(2):
        o0_ref[side] = val[:, side * F:(side + 1) * F]
        o1_ref[side] = val[:, (2 + side) * F:(3 + side) * F]


def _node_proj(h, w_cat, block_rows=2000):
    n, f = h.shape
    return pl.pallas_call(
        _proj_kernel,
        grid=(n // block_rows,),
        in_specs=[
            pl.BlockSpec((block_rows, f), lambda i: (i, 0)),
            pl.BlockSpec((f, 4 * F), lambda i: (0, 0)),
        ],
        out_specs=[pl.BlockSpec((2, block_rows, F), lambda i: (0, i, 0)),
                   pl.BlockSpec((2, block_rows, F), lambda i: (0, i, 0))],
        out_shape=[jax.ShapeDtypeStruct((2, n, F), jnp.float32),
                   jax.ShapeDtypeStruct((2, n, F), jnp.float32)],
    )(h, w_cat)


def _bn_stats_kernel(a0_ref, a1_ref, o_ref):
    @pl.when(pl.program_id(0) == 0)
    def _():
        o_ref[...] = jnp.zeros_like(o_ref)

    x0 = a0_ref[...]
    x1 = a1_ref[...]
    o_ref[...] += jnp.concatenate(
        [jnp.sum(x0, axis=0, keepdims=True),
         jnp.sum(x1, axis=0, keepdims=True),
         jnp.sum(x0 * x0, axis=0, keepdims=True),
         jnp.sum(x1 * x1, axis=0, keepdims=True)], axis=0)


def _bn_apply_kernel(stats_ref, a0_ref, a1_ref, h_ref, g_ref, b_ref, o_ref,
                     *, n):
    s = stats_ref[...]
    mean = s[0:2, :] / n
    var = s[2:4, :] / n - mean * mean
    rstd = lax.rsqrt(var + 1e-5)
    g = g_ref[...]
    b = b_ref[...]
    h = h_ref[...]
    lo = (a0_ref[...] - mean[0:1]) * (rstd[0:1] * g[0:1]) + b[0:1] \
        + h[:, :FH]
    hi = (a1_ref[...] - mean[1:2]) * (rstd[1:2] * g[1:2]) + b[1:2] \
        + h[:, FH:]
    o_ref[...] = jnp.concatenate([lo, hi], axis=1)


def _bn_residual(agg_q0, agg_q1, h, gamma, beta, block_rows=2000):
    n, f = h.shape
    stats = pl.pallas_call(
        _bn_stats_kernel,
        grid=(n // block_rows,),
        in_specs=[pl.BlockSpec((block_rows, FH), lambda i: (i, 0)),
                  pl.BlockSpec((block_rows, FH), lambda i: (i, 0))],
        out_specs=pl.BlockSpec((4, FH), lambda i: (0, 0)),
        out_shape=jax.ShapeDtypeStruct((4, FH), jnp.float32),
    )(agg_q0, agg_q1)
    return pl.pallas_call(
        functools.partial(_bn_apply_kernel, n=float(n)),
        grid=(n // block_rows,),
        in_specs=[
            pl.BlockSpec((4, FH), lambda i: (0, 0)),
            pl.BlockSpec((block_rows, FH), lambda i: (i, 0)),
            pl.BlockSpec((block_rows, FH), lambda i: (i, 0)),
            pl.BlockSpec((block_rows, f), lambda i: (i, 0)),
            pl.BlockSpec((2, FH), lambda i: (0, 0)),
            pl.BlockSpec((2, FH), lambda i: (0, 0)),
        ],
        out_specs=pl.BlockSpec((block_rows, f), lambda i: (i, 0)),
        out_shape=jax.ShapeDtypeStruct((n, f), jnp.float32),
    )(stats, agg_q0, agg_q1, h, gamma.reshape(2, FH), beta.reshape(2, FH))


def _pool_kernel(h_ref, batch_ref, gw1_ref, gb1_ref, gw2_ref, gb2_ref,
                 nw1_ref, nb1_ref, nw2_ref, nb2_ref,
                 h1w_ref, h1b_ref, h2w_ref, h2b_ref, ow_ref, ob_ref, o_ref,
                 *, num_graphs):
    h = h_ref[...]
    g = (jnp.dot(jnp.maximum(
        jnp.dot(h, gw1_ref[...], preferred_element_type=jnp.float32)
        + gb1_ref[...], 0.0), gw2_ref[...],
        preferred_element_type=jnp.float32) + gb2_ref[...])
    g = g[:, 0:1]
    t = (jnp.dot(jnp.maximum(
        jnp.dot(h, nw1_ref[...], preferred_element_type=jnp.float32)
        + nb1_ref[...], 0.0), nw2_ref[...],
        preferred_element_type=jnp.float32) + nb2_ref[...])
    bcol = batch_ref[...]
    seg = lax.broadcasted_iota(jnp.int32, (1, num_graphs), 1)
    msk = (bcol == seg)
    gmax = jnp.max(jnp.where(msk, g, -jnp.inf), axis=0, keepdims=True)
    gnode = jnp.sum(jnp.where(msk, gmax, 0.0), axis=1, keepdims=True)
    gexp = jnp.exp(g - gnode)
    mskf = msk.astype(jnp.float32)
    gsum = lax.dot_general(mskf, gexp, (((0,), (0,)), ((), ())),
                           preferred_element_type=jnp.float32)
    st = lax.dot_general(mskf, gexp * t, (((0,), (0,)), ((), ())),
                         preferred_element_type=jnp.float32)
    out = st / gsum
    out = _lrelu(jnp.dot(out, h1w_ref[...],
                         preferred_element_type=jnp.float32) + h1b_ref[...])
    out = _lrelu(jnp.dot(out, h2w_ref[...],
                         preferred_element_type=jnp.float32) + h2b_ref[...])
    o_ref[...] = (jnp.dot(out, ow_ref[...],
                          preferred_element_type=jnp.float32) + ob_ref[...])


def _pool(h, batch, params, num_graphs):
    n, f = h.shape
    gw2 = jnp.pad(params['gate_W2'], ((0, 0), (0, 127)))
    gb2 = jnp.pad(params['gate_b2'].reshape(1, 1), ((0, 0), (0, 127)))
    ow = jnp.pad(params['W_out'], ((0, 0), (0, 127)))
    ob = jnp.pad(params['b_out'].reshape(1, 1), ((0, 0), (0, 127)))
    out = pl.pallas_call(
        functools.partial(_pool_kernel, num_graphs=num_graphs),
        in_specs=[
            pl.BlockSpec((n, f), lambda: (0, 0)),
            pl.BlockSpec((n, 1), lambda: (0, 0)),
            pl.BlockSpec((f, f), lambda: (0, 0)),
            pl.BlockSpec((1, f), lambda: (0, 0)),
            pl.BlockSpec((f, 128), lambda: (0, 0)),
            pl.BlockSpec((1, 128), lambda: (0, 0)),
            pl.BlockSpec((f, f), lambda: (0, 0)),
            pl.BlockSpec((1, f), lambda: (0, 0)),
            pl.BlockSpec((f, f), lambda: (0, 0)),
            pl.BlockSpec((1, f), lambda: (0, 0)),
            pl.BlockSpec((f, f), lambda: (0, 0)),
            pl.BlockSpec((1, f), lambda: (0, 0)),
            pl.BlockSpec((f, f), lambda: (0, 0)),
            pl.BlockSpec((1, f), lambda: (0, 0)),
            pl.BlockSpec((f, 128), lambda: (0, 0)),
            pl.BlockSpec((1, 128), lambda: (0, 0)),
        ],
        out_specs=pl.BlockSpec((num_graphs, 128), lambda: (0, 0)),
        out_shape=jax.ShapeDtypeStruct((num_graphs, 128), jnp.float32),
    )(h, batch.reshape(n, 1).astype(jnp.int32),
      params['gate_W1'], params['gate_b1'].reshape(1, f), gw2, gb2,
      params['nn_W1'], params['nn_b1'].reshape(1, f),
      params['nn_W2'], params['nn_b2'].reshape(1, f),
      params['heads'][0]['W'], params['heads'][0]['b'].reshape(1, f),
      params['heads'][1]['W'], params['heads'][1]['b'].reshape(1, f),
      ow, ob)
    return out[:, 0]


_SC_C = 48
_SC_SUP = 32


def _softplus_sc(zs):
    u = jnp.exp(-jnp.abs(zs))
    t = u / (u + 2.0)
    t2 = t * t
    p = 1.0 + t2 * (1.0 / 3.0 + t2 * 0.2)
    return jnp.maximum(zs, 0.0) + 2.0 * t * p


def _sigmoid_sc(zf):
    u = jnp.exp(-jnp.abs(zf))
    return jnp.where(zf >= 0, 1.0, u) / (1.0 + u)


def _make_sc_pass(n, nh, NH, ntc):
    C = _SC_C
    SUP = _SC_SUP
    mesh = plsc.VectorSubcoreMesh(core_axis_name="c", subcore_axis_name="s")
    rpt = NH // 16
    assert rpt % 8 == 0 and ntc % SUP == 0

    @functools.partial(
        pl.kernel, mesh=mesh,
        out_type=jax.ShapeDtypeStruct((2 * NH, FH), jnp.float32),
        scratch_types=[
            pltpu.VMEM((SUP, 128), jnp.int32),
            pltpu.VMEM((C,), jnp.int32),
            pltpu.VMEM((C, F), jnp.float32),
            pltpu.VMEM((C, F), jnp.float32),
            pltpu.VMEM((C, F), jnp.float32),
            pltpu.VMEM((C, F), jnp.float32),
            pltpu.VMEM((C, F), jnp.float32),
            pltpu.VMEM((C, F), jnp.float32),
            pltpu.VMEM((C, FH), jnp.float32),
            pltpu.VMEM_SHARED((NH, FH), jnp.float32),
            pltpu.SemaphoreType.DMA,
            pltpu.SemaphoreType.DMA,
            pltpu.SemaphoreType.DMA,
            pltpu.SemaphoreType.DMA,
            pltpu.SemaphoreType.DMA,
            pltpu.SemaphoreType.DMA,
        ],
    )
    def sc_pass(t_hbm, ea_hbm, sd_hbm, z_hbm, out_hbm,
                sd_v, sc_i, gd0, gd1, gs0, gs1, ea0, ea1, m_v, slab,
                smd0, smd1, sms0, sms1, sme0, sme1):
        c = lax.axis_index("c")
        s = lax.axis_index("s")
        pltpu.sync_copy(z_hbm.at[pl.ds(s * rpt, rpt), :],
                        slab.at[pl.ds(s * rpt, rpt), :])
        plsc.subcore_barrier()

        cnh = c * nh
        base = s * ntc

        def issue(j, r, gd, gs, ea, sd_, ss_, se_):
            pltpu.async_copy(t_hbm.at[sd_v.at[r, pl.ds(0, C)]], gd, sd_)
            pltpu.async_copy(t_hbm.at[sd_v.at[r, pl.ds(C, C)]], gs, ss_)
            pltpu.async_copy(
                ea_hbm.at[pl.ds(pl.multiple_of((base + j) * C, 16), C), :],
                ea, se_)

        def work(r, gd, gs, ea, sd_, ss_, se_):
            pltpu.make_async_copy(t_hbm.at[pl.ds(0, C), :], gd, sd_).wait()
            pltpu.make_async_copy(t_hbm.at[pl.ds(0, C), :], gs, ss_).wait()
            pltpu.make_async_copy(ea_hbm.at[pl.ds(0, C), :], ea, se_).wait()
            for k in range(C // 16):
                sl = pl.ds(k * 16, 16)
                dl = sd_v[r, sl] - cnh
                inb = jnp.logical_and(dl >= 0, dl < nh)
                sc_i[sl] = jnp.where(inb, dl, nh)

            def row(j, carry2):
                for k in range(FH // 16):
                    slf = pl.ds(k * 16, 16)
                    sls = pl.ds(FH + k * 16, 16)
                    zf = gd[j, slf] + gs[j, slf] + ea[j, slf]
                    zs = gd[j, sls] + gs[j, sls] + ea[j, sls]
                    m_v[j, slf] = _sigmoid_sc(zf) * _softplus_sc(zs)
                return carry2

            lax.fori_loop(0, C, row, 0)
            pltpu.sync_copy(m_v, slab.at[sc_i], add=True)

        pltpu.sync_copy(
            sd_hbm.at[pl.ds(pl.multiple_of(base, SUP), SUP), :], sd_v)
        issue(0, 0, gd0, gs0, ea0, smd0, sms0, sme0)

        def body(i, carry):
            r = lax.rem(i, SUP)
            even = lax.rem(i, 2) == 0

            @pl.when(even)
            def _():
                work(r, gd0, gs0, ea0, smd0, sms0, sme0)

            @pl.when(jnp.logical_not(even))
            def _():
                work(r, gd1, gs1, ea1, smd1, sms1, sme1)

            rn = lax.rem(i + 1, SUP)
            more = i + 1 < ntc

            @pl.when(jnp.logical_and(rn == 0, more))
            def _():
                pltpu.sync_copy(
                    sd_hbm.at[pl.ds(pl.multiple_of(base + i + 1, SUP), SUP),
                              :], sd_v)

            @pl.when(jnp.logical_and(more, even))
            def _():
                issue(i + 1, rn, gd1, gs1, ea1, smd1, sms1, sme1)

            @pl.when(jnp.logical_and(more, jnp.logical_not(even)))
            def _():
                issue(i + 1, rn, gd0, gs0, ea0, smd0, sms0, sme0)

            return carry

        lax.fori_loop(0, ntc, body, 0)
        plsc.subcore_barrier()
        pltpu.sync_copy(slab.at[pl.ds(s * rpt, rpt), :],
                        out_hbm.at[pl.ds(c * NH + s * rpt, rpt), :])

    return sc_pass


def kernel(x, edge_attr, params, edge_index, batch):
    n, node_in = x.shape
    e, edge_in = edge_attr.shape
    num_graphs = 64
    num_layers = len(params['convs'])

    src = edge_index[0].astype(jnp.int32)
    dst = edge_index[1].astype(jnp.int32)
    n_pad = ((n + 127) // 128) * 128
    nh = n_pad // 2
    NH = ((nh + 128) // 128) * 128
    zeros = jnp.zeros((NH, FH), jnp.float32)

    C = _SC_C
    ntc = -(-e // (16 * C))
    ntc = ((ntc + _SC_SUP - 1) // _SC_SUP) * _SC_SUP
    e_pad = 16 * C * ntc
    dst_p = jnp.concatenate([dst, jnp.full((e_pad - e,), n, jnp.int32)])
    src_p = jnp.concatenate([src, jnp.zeros((e_pad - e,), jnp.int32)])
    ea_in = jnp.concatenate(
        [edge_attr, jnp.zeros((e_pad - e, edge_in), jnp.float32)])
    nch = e_pad // C
    sd = jnp.concatenate(
        [dst_p.reshape(nch, C), src_p.reshape(nch, C) + n,
         jnp.zeros((nch, 128 - 2 * C), jnp.int32)], axis=1)

    kp = 256
    x_p = jnp.pad(x, ((0, 0), (0, kp - node_in)))
    wn_p = jnp.pad(params['W_node'], ((0, kp - node_in), (0, 0)))
    h = _mm_bias_act(x_p, wn_p, params['b_node'].reshape(1, F), _lrelu, 2000)

    kpe = 16
    ea_p = jnp.pad(ea_in, ((0, 0), (0, kpe - edge_in)))
    we_p = jnp.pad(params['W_edge'], ((0, kpe - edge_in), (0, 0)))
    wcols, bcols = [], []
    for p in params['convs']:
        for q in range(2):
            sl = slice(q * FH, (q + 1) * FH)
            wcols += [p['Wf'][2 * F:, sl], p['Ws'][2 * F:, sl]]
            bcols += [p['bf'][sl], p['bs'][sl]]
    w_all = jnp.concatenate(wcols, axis=1)
    b_all = jnp.concatenate(bcols)
    ea_passes = _edge_precompute(ea_p, we_p, params['b_edge'].reshape(1, F),
                                 w_all, b_all.reshape(1, -1), 2 * num_layers,
                                 block_rows=512)

    sc_pass = _make_sc_pass(n, nh, NH, ntc)
    for li, p in enumerate(params['convs']):
        cols = []
        for q in range(2):
            sl = slice(q * FH, (q + 1) * FH)
            for half in range(2):
                cols += [p['Wf'][half * F:(half + 1) * F, sl],
                         p['Ws'][half * F:(half + 1) * F, sl]]
        w_cat = jnp.concatenate(cols, axis=1)
        t_q = _node_proj(h, w_cat)
        aggs = []
        for q in range(2):
            o = sc_pass(t_q[q].reshape(2 * n, F), ea_passes[li * 2 + q],
                        sd, zeros)
            aggs.append(jnp.concatenate([o[:nh], o[NH:NH + n - nh]], axis=0))
        h = _bn_residual(aggs[0], aggs[1], h, p['gamma'], p['beta'])

    return _pool(h, batch, params, num_graphs)


# speedup vs baseline: 1.1964x; 1.1964x over previous
import functools

import jax
import jax.numpy as jnp
from jax import lax
from jax.experimental import pallas as pl
from jax.experimental.pallas import tpu as pltpu
from jax.experimental.pallas import tpu_sc as plsc

F = 256
FH = 128


def _lrelu(t):
    return jnp.where(t >= 0, t, 0.01 * t)


def _mm_bias_act_kernel(x_ref, w_ref, b_ref, o_ref, *, act):
    o_ref[...] = act(
        jnp.dot(x_ref[...], w_ref[...], preferred_element_type=jnp.float32)
        + b_ref[...])


def _mm_bias_act(x, w, b, act, block_rows):
    rows, k = x.shape
    m = w.shape[1]
    return pl.pallas_call(
        functools.partial(_mm_bias_act_kernel, act=act),
        grid=(rows // block_rows,),
        in_specs=[
            pl.BlockSpec((block_rows, k), lambda i: (i, 0)),
            pl.BlockSpec((k, m), lambda i: (0, 0)),
            pl.BlockSpec((1, m), lambda i: (0, 0)),
        ],
        out_specs=pl.BlockSpec((block_rows, m), lambda i: (i, 0)),
        out_shape=jax.ShapeDtypeStruct((rows, m), jnp.float32),
    )(x, w, b)


def _edge_precompute_kernel(ea_ref, we_ref, be_ref, wall_ref, ball_ref,
                            *o_refs, nplanes):
    ea = _lrelu(
        jnp.dot(ea_ref[...], we_ref[...], preferred_element_type=jnp.float32)
        + be_ref[...])
    val = (jnp.dot(ea, wall_ref[...], preferred_element_type=jnp.float32)
           + ball_ref[...])
    for j in range(nplanes):
        o_refs[j][...] = val[:, j * F:(j + 1) * F]


def _edge_precompute(edge_attr_p, w_e, b_e, w_all, b_all, nplanes,
                     block_rows=640):
    e, k = edge_attr_p.shape
    m = w_all.shape[1]
    return pl.pallas_call(
        functools.partial(_edge_precompute_kernel, nplanes=nplanes),
        grid=(e // block_rows,),
        in_specs=[
            pl.BlockSpec((block_rows, k), lambda i: (i, 0)),
            pl.BlockSpec((k, F), lambda i: (0, 0)),
            pl.BlockSpec((1, F), lambda i: (0, 0)),
            pl.BlockSpec((F, m), lambda i: (0, 0)),
            pl.BlockSpec((1, m), lambda i: (0, 0)),
        ],
        out_specs=[pl.BlockSpec((block_rows, F), lambda i: (i, 0))
                   for _ in range(nplanes)],
        out_shape=[jax.ShapeDtypeStruct((e, F), jnp.float32)
                   for _ in range(nplanes)],
    )(edge_attr_p, w_e, b_e, w_all, b_all)


def _proj_kernel(h_ref, w_ref, o0_ref, o1_ref):
    val = jnp.dot(h_ref[...], w_ref[...], preferred_element_type=jnp.float32)
    for side in range(2):
        o0_ref[side] = val[:, side * F:(side + 1) * F]
        o1_ref[side] = val[:, (2 + side) * F:(3 + side) * F]


def _node_proj(h, w_cat, block_rows=2000):
    n, f = h.shape
    return pl.pallas_call(
        _proj_kernel,
        grid=(n // block_rows,),
        in_specs=[
            pl.BlockSpec((block_rows, f), lambda i: (i, 0)),
            pl.BlockSpec((f, 4 * F), lambda i: (0, 0)),
        ],
        out_specs=[pl.BlockSpec((2, block_rows, F), lambda i: (0, i, 0)),
                   pl.BlockSpec((2, block_rows, F), lambda i: (0, i, 0))],
        out_shape=[jax.ShapeDtypeStruct((2, n, F), jnp.float32),
                   jax.ShapeDtypeStruct((2, n, F), jnp.float32)],
    )(h, w_cat)


def _bn_stats_kernel(a0_ref, a1_ref, o_ref):
    @pl.when(pl.program_id(0) == 0)
    def _():
        o_ref[...] = jnp.zeros_like(o_ref)

    x0 = a0_ref[...]
    x1 = a1_ref[...]
    o_ref[...] += jnp.concatenate(
        [jnp.sum(x0, axis=0, keepdims=True),
         jnp.sum(x1, axis=0, keepdims=True),
         jnp.sum(x0 * x0, axis=0, keepdims=True),
         jnp.sum(x1 * x1, axis=0, keepdims=True)], axis=0)


def _bn_apply_kernel(stats_ref, a0_ref, a1_ref, h_ref, g_ref, b_ref, o_ref,
                     *, n):
    s = stats_ref[...]
    mean = s[0:2, :] / n
    var = s[2:4, :] / n - mean * mean
    rstd = lax.rsqrt(var + 1e-5)
    g = g_ref[...]
    b = b_ref[...]
    h = h_ref[...]
    lo = (a0_ref[...] - mean[0:1]) * (rstd[0:1] * g[0:1]) + b[0:1] \
        + h[:, :FH]
    hi = (a1_ref[...] - mean[1:2]) * (rstd[1:2] * g[1:2]) + b[1:2] \
        + h[:, FH:]
    o_ref[...] = jnp.concatenate([lo, hi], axis=1)


def _bn_residual(agg_q0, agg_q1, h, gamma, beta, block_rows=2000):
    n, f = h.shape
    stats = pl.pallas_call(
        _bn_stats_kernel,
        grid=(n // block_rows,),
        in_specs=[pl.BlockSpec((block_rows, FH), lambda i: (i, 0)),
                  pl.BlockSpec((block_rows, FH), lambda i: (i, 0))],
        out_specs=pl.BlockSpec((4, FH), lambda i: (0, 0)),
        out_shape=jax.ShapeDtypeStruct((4, FH), jnp.float32),
    )(agg_q0, agg_q1)
    return pl.pallas_call(
        functools.partial(_bn_apply_kernel, n=float(n)),
        grid=(n // block_rows,),
        in_specs=[
            pl.BlockSpec((4, FH), lambda i: (0, 0)),
            pl.BlockSpec((block_rows, FH), lambda i: (i, 0)),
            pl.BlockSpec((block_rows, FH), lambda i: (i, 0)),
            pl.BlockSpec((block_rows, f), lambda i: (i, 0)),
            pl.BlockSpec((2, FH), lambda i: (0, 0)),
            pl.BlockSpec((2, FH), lambda i: (0, 0)),
        ],
        out_specs=pl.BlockSpec((block_rows, f), lambda i: (i, 0)),
        out_shape=jax.ShapeDtypeStruct((n, f), jnp.float32),
    )(stats, agg_q0, agg_q1, h, gamma.reshape(2, FH), beta.reshape(2, FH))


def _pool_kernel(h_ref, batch_ref, gw1_ref, gb1_ref, gw2_ref, gb2_ref,
                 nw1_ref, nb1_ref, nw2_ref, nb2_ref,
                 h1w_ref, h1b_ref, h2w_ref, h2b_ref, ow_ref, ob_ref, o_ref,
                 *, num_graphs):
    h = h_ref[...]
    g = (jnp.dot(jnp.maximum(
        jnp.dot(h, gw1_ref[...], preferred_element_type=jnp.float32)
        + gb1_ref[...], 0.0), gw2_ref[...],
        preferred_element_type=jnp.float32) + gb2_ref[...])
    g = g[:, 0:1]
    t = (jnp.dot(jnp.maximum(
        jnp.dot(h, nw1_ref[...], preferred_element_type=jnp.float32)
        + nb1_ref[...], 0.0), nw2_ref[...],
        preferred_element_type=jnp.float32) + nb2_ref[...])
    bcol = batch_ref[...]
    seg = lax.broadcasted_iota(jnp.int32, (1, num_graphs), 1)
    msk = (bcol == seg)
    gmax = jnp.max(jnp.where(msk, g, -jnp.inf), axis=0, keepdims=True)
    gnode = jnp.sum(jnp.where(msk, gmax, 0.0), axis=1, keepdims=True)
    gexp = jnp.exp(g - gnode)
    mskf = msk.astype(jnp.float32)
    gsum = lax.dot_general(mskf, gexp, (((0,), (0,)), ((), ())),
                           preferred_element_type=jnp.float32)
    st = lax.dot_general(mskf, gexp * t, (((0,), (0,)), ((), ())),
                         preferred_element_type=jnp.float32)
    out = st / gsum
    out = _lrelu(jnp.dot(out, h1w_ref[...],
                         preferred_element_type=jnp.float32) + h1b_ref[...])
    out = _lrelu(jnp.dot(out, h2w_ref[...],
                         preferred_element_type=jnp.float32) + h2b_ref[...])
    o_ref[...] = (jnp.dot(out, ow_ref[...],
                          preferred_element_type=jnp.float32) + ob_ref[...])


def _pool(h, batch, params, num_graphs):
    n, f = h.shape
    gw2 = jnp.pad(params['gate_W2'], ((0, 0), (0, 127)))
    gb2 = jnp.pad(params['gate_b2'].reshape(1, 1), ((0, 0), (0, 127)))
    ow = jnp.pad(params['W_out'], ((0, 0), (0, 127)))
    ob = jnp.pad(params['b_out'].reshape(1, 1), ((0, 0), (0, 127)))
    out = pl.pallas_call(
        functools.partial(_pool_kernel, num_graphs=num_graphs),
        in_specs=[
            pl.BlockSpec((n, f), lambda: (0, 0)),
            pl.BlockSpec((n, 1), lambda: (0, 0)),
            pl.BlockSpec((f, f), lambda: (0, 0)),
            pl.BlockSpec((1, f), lambda: (0, 0)),
            pl.BlockSpec((f, 128), lambda: (0, 0)),
            pl.BlockSpec((1, 128), lambda: (0, 0)),
            pl.BlockSpec((f, f), lambda: (0, 0)),
            pl.BlockSpec((1, f), lambda: (0, 0)),
            pl.BlockSpec((f, f), lambda: (0, 0)),
            pl.BlockSpec((1, f), lambda: (0, 0)),
            pl.BlockSpec((f, f), lambda: (0, 0)),
            pl.BlockSpec((1, f), lambda: (0, 0)),
            pl.BlockSpec((f, f), lambda: (0, 0)),
            pl.BlockSpec((1, f), lambda: (0, 0)),
            pl.BlockSpec((f, 128), lambda: (0, 0)),
            pl.BlockSpec((1, 128), lambda: (0, 0)),
        ],
        out_specs=pl.BlockSpec((num_graphs, 128), lambda: (0, 0)),
        out_shape=jax.ShapeDtypeStruct((num_graphs, 128), jnp.float32),
    )(h, batch.reshape(n, 1).astype(jnp.int32),
      params['gate_W1'], params['gate_b1'].reshape(1, f), gw2, gb2,
      params['nn_W1'], params['nn_b1'].reshape(1, f),
      params['nn_W2'], params['nn_b2'].reshape(1, f),
      params['heads'][0]['W'], params['heads'][0]['b'].reshape(1, f),
      params['heads'][1]['W'], params['heads'][1]['b'].reshape(1, f),
      ow, ob)
    return out[:, 0]


_SC_C = 80
_SC_SUP = 32


def _softplus_sc(zs):
    u = jnp.exp(-jnp.abs(zs))
    t = u / (u + 2.0)
    t2 = t * t
    p = 1.0 + t2 * (1.0 / 3.0 + t2 * 0.2)
    return jnp.maximum(zs, 0.0) + 2.0 * t * p


def _sigmoid_sc(zf):
    u = jnp.exp(-jnp.abs(zf))
    return jnp.where(zf >= 0, 1.0, u) / (1.0 + u)


def _make_sc_pass(n, nh, NH, ntc):
    C = _SC_C
    SUP = _SC_SUP
    mesh = plsc.VectorSubcoreMesh(core_axis_name="c", subcore_axis_name="s")
    rpt = NH // 16
    assert rpt % 8 == 0 and ntc % SUP == 0

    @functools.partial(
        pl.kernel, mesh=mesh,
        out_type=jax.ShapeDtypeStruct((2 * NH, FH), jnp.float32),
        scratch_types=[
            pltpu.VMEM((SUP, C), jnp.int32),
            pltpu.VMEM((SUP, C), jnp.int32),
            pltpu.VMEM((C,), jnp.int32),
            pltpu.VMEM((C, F), jnp.float32),
            pltpu.VMEM((C, F), jnp.float32),
            pltpu.VMEM((C, F), jnp.float32),
            pltpu.VMEM((C, FH), jnp.float32),
            pltpu.VMEM_SHARED((NH, FH), jnp.float32),
            pltpu.SemaphoreType.DMA,
            pltpu.SemaphoreType.DMA,
            pltpu.SemaphoreType.DMA,
        ],
    )
    def sc_pass(t_hbm, ea_hbm, sdd_hbm, sds_hbm, z_hbm, out_hbm,
                sdd_v, sds_v, sc_i, gd_v, gs_v, ea_v, m_v, slab,
                sm0, sm1, sm2):
        c = lax.axis_index("c")
        s = lax.axis_index("s")
        pltpu.sync_copy(z_hbm.at[pl.ds(s * rpt, rpt), :],
                        slab.at[pl.ds(s * rpt, rpt), :])
        plsc.subcore_barrier()

        cnh = c * nh
        base = s * ntc

        def body(i, carry):
            r = lax.rem(i, SUP)

            @pl.when(r == 0)
            def _():
                pltpu.sync_copy(
                    sdd_hbm.at[pl.ds(pl.multiple_of(base + i, SUP), SUP), :],
                    sdd_v)
                pltpu.sync_copy(
                    sds_hbm.at[pl.ds(pl.multiple_of(base + i, SUP), SUP), :],
                    sds_v)

            cp0 = pltpu.async_copy(t_hbm.at[sdd_v.at[r]], gd_v, sm0)
            cp1 = pltpu.async_copy(t_hbm.at[sds_v.at[r]], gs_v, sm1)
            cp2 = pltpu.async_copy(
                ea_hbm.at[pl.ds(pl.multiple_of((base + i) * C, 16), C), :],
                ea_v, sm2)
            for k in range(C // 16):
                sl = pl.ds(k * 16, 16)
                dl = sdd_v[r, sl] - cnh
                inb = jnp.logical_and(dl >= 0, dl < nh)
                sc_i[sl] = jnp.where(inb, dl, nh)
            cp0.wait()
            cp1.wait()
            cp2.wait()

            def row(j, carry2):
                for k in range(FH // 16):
                    slf = pl.ds(k * 16, 16)
                    sls = pl.ds(FH + k * 16, 16)
                    zf = gd_v[j, slf] + gs_v[j, slf] + ea_v[j, slf]
                    zs = gd_v[j, sls] + gs_v[j, sls] + ea_v[j, sls]
                    m_v[j, slf] = _sigmoid_sc(zf) * _softplus_sc(zs)
                return carry2

            lax.fori_loop(0, C, row, 0)
            pltpu.sync_copy(m_v, slab.at[sc_i], add=True)
            return carry

        lax.fori_loop(0, ntc, body, 0)
        plsc.subcore_barrier()
        pltpu.sync_copy(slab.at[pl.ds(s * rpt, rpt), :],
                        out_hbm.at[pl.ds(c * NH + s * rpt, rpt), :])

    return sc_pass


def kernel(x, edge_attr, params, edge_index, batch):
    n, node_in = x.shape
    e, edge_in = edge_attr.shape
    num_graphs = 64
    num_layers = len(params['convs'])

    src = edge_index[0].astype(jnp.int32)
    dst = edge_index[1].astype(jnp.int32)
    n_pad = ((n + 127) // 128) * 128
    nh = n_pad // 2
    NH = ((nh + 128) // 128) * 128
    zeros = jnp.zeros((NH, FH), jnp.float32)

    C = _SC_C
    ntc = -(-e // (16 * C))
    ntc = ((ntc + _SC_SUP - 1) // _SC_SUP) * _SC_SUP
    e_pad = 16 * C * ntc
    dst_p = jnp.concatenate([dst, jnp.full((e_pad - e,), n, jnp.int32)])
    src_p = jnp.concatenate([src, jnp.zeros((e_pad - e,), jnp.int32)])
    ea_in = jnp.concatenate(
        [edge_attr, jnp.zeros((e_pad - e, edge_in), jnp.float32)])
    nch = e_pad // C
    sdd = dst_p.reshape(nch, C)
    sds = src_p.reshape(nch, C) + n

    kp = 256
    x_p = jnp.pad(x, ((0, 0), (0, kp - node_in)))
    wn_p = jnp.pad(params['W_node'], ((0, kp - node_in), (0, 0)))
    h = _mm_bias_act(x_p, wn_p, params['b_node'].reshape(1, F), _lrelu, 2000)

    kpe = 16
    ea_p = jnp.pad(ea_in, ((0, 0), (0, kpe - edge_in)))
    we_p = jnp.pad(params['W_edge'], ((0, kpe - edge_in), (0, 0)))
    wcols, bcols = [], []
    for p in params['convs']:
        for q in range(2):
            sl = slice(q * FH, (q + 1) * FH)
            wcols += [p['Wf'][2 * F:, sl], p['Ws'][2 * F:, sl]]
            bcols += [p['bf'][sl], p['bs'][sl]]
    w_all = jnp.concatenate(wcols, axis=1)
    b_all = jnp.concatenate(bcols)
    ea_passes = _edge_precompute(ea_p, we_p, params['b_edge'].reshape(1, F),
                                 w_all, b_all.reshape(1, -1), 2 * num_layers,
                                 block_rows=512)

    sc_pass = _make_sc_pass(n, nh, NH, ntc)
    for li, p in enumerate(params['convs']):
        cols = []
        for q in range(2):
            sl = slice(q * FH, (q + 1) * FH)
            for half in range(2):
                cols += [p['Wf'][half * F:(half + 1) * F, sl],
                         p['Ws'][half * F:(half + 1) * F, sl]]
        w_cat = jnp.concatenate(cols, axis=1)
        t_q = _node_proj(h, w_cat)
        aggs = []
        for q in range(2):
            o = sc_pass(t_q[q].reshape(2 * n, F), ea_passes[li * 2 + q],
                        sdd, sds, zeros)
            aggs.append(jnp.concatenate([o[:nh], o[NH:NH + n - nh]], axis=0))
        h = _bn_residual(aggs[0], aggs[1], h, p['gamma'], p['beta'])

    return _pool(h, batch, params, num_graphs)
